# baseline (speedup 1.0000x reference)
import functools

import jax
import jax.numpy as jnp
from jax import lax
from jax.experimental import pallas as pl
from jax.experimental.pallas import tpu as pltpu
from jax.experimental.pallas import tpu_sc as plsc

HIDDEN = 128
E = 320000
VOCAB = 100
NUM_CORES = 2
NUM_SUBCORES = 16
NUM_WORKERS = NUM_CORES * NUM_SUBCORES
PER_WORKER = E // NUM_WORKERS
CHUNK = 80
NUM_CHUNKS = PER_WORKER // CHUNK
LANES = 16

_mesh = plsc.VectorSubcoreMesh(core_axis_name="c", subcore_axis_name="s")


@functools.partial(
    pl.kernel,
    mesh=_mesh,
    out_type=jax.ShapeDtypeStruct((E, HIDDEN), jnp.float32),
    scratch_types=[
        pltpu.VMEM((PER_WORKER,), jnp.int32),
        pltpu.VMEM((PER_WORKER,), jnp.int32),
        pltpu.VMEM((PER_WORKER,), jnp.int32),
        pltpu.VMEM((CHUNK,), jnp.int32),
        pltpu.VMEM((CHUNK,), jnp.int32),
        pltpu.VMEM((CHUNK, HIDDEN), jnp.float32),
        pltpu.VMEM((CHUNK, HIDDEN), jnp.float32),
        pltpu.VMEM((CHUNK, HIDDEN), jnp.float32),
        pltpu.VMEM((CHUNK, HIDDEN), jnp.float32),
        pltpu.VMEM((CHUNK, HIDDEN), jnp.float32),
        pltpu.VMEM((CHUNK, HIDDEN), jnp.float32),
        pltpu.VMEM_SHARED((3 * VOCAB, HIDDEN), jnp.float32),
        pltpu.VMEM_SHARED((2 * NUM_SUBCORES * CHUNK, HIDDEN), jnp.float32),
        pltpu.SemaphoreType.DMA,
        pltpu.SemaphoreType.DMA,
        pltpu.SemaphoreType.DMA,
        pltpu.SemaphoreType.DMA,
        pltpu.SemaphoreType.DMA,
        pltpu.SemaphoreType.DMA,
        pltpu.SemaphoreType.DMA,
        pltpu.SemaphoreType.DMA,
    ],
)
def _bond_encoder_sc(i0_hbm, i1_hbm, i2_hbm, t0_hbm, t1_hbm, t2_hbm,
                     out_hbm, i0_v, i1_v, i2_v, ids0_v, ids1_v,
                     a0_v, a1_v, a2_v, c0_v, c1_v, c2_v,
                     tab_sh, acc_sh, ga0, ga1, ga2, gb0, gb1, gb2, w0, w1):
    sid = lax.axis_index("s")
    wid = sid * NUM_CORES + lax.axis_index("c")
    base = wid * PER_WORKER

    bufs = ((a0_v, a1_v, a2_v), (c0_v, c1_v, c2_v))
    gsems = ((ga0, ga1, ga2), (gb0, gb1, gb2))
    wsems = (w0, w1)
    idx_v = (i0_v, i1_v, i2_v)
    ids_v = (ids0_v, ids1_v)

    @pl.when(sid == 0)
    def _stage():
        pltpu.sync_copy(t0_hbm, tab_sh.at[pl.ds(0, VOCAB)])
        pltpu.sync_copy(t1_hbm, tab_sh.at[pl.ds(VOCAB, VOCAB)])
        pltpu.sync_copy(t2_hbm, tab_sh.at[pl.ds(2 * VOCAB, VOCAB)])

    pltpu.sync_copy(i0_hbm.at[pl.ds(base, PER_WORKER)], i0_v)
    pltpu.sync_copy(i1_hbm.at[pl.ds(base, PER_WORKER)], i1_v)
    pltpu.sync_copy(i2_hbm.at[pl.ds(base, PER_WORKER)], i2_v)

    def rebase(j, carry):
        sl = pl.ds(j * LANES, LANES)
        i1_v[sl] = i1_v[sl] + VOCAB
        i2_v[sl] = i2_v[sl] + 2 * VOCAB
        return carry

    lax.fori_loop(0, PER_WORKER // LANES, rebase, 0)

    def build_ids(j, carry):
        sl = pl.ds(j * LANES, LANES)
        lane = lax.iota(jnp.int32, LANES) + j * LANES
        ids0_v[sl] = lane + (2 * sid) * CHUNK
        ids1_v[sl] = lane + (2 * sid + 1) * CHUNK
        return carry

    lax.fori_loop(0, CHUNK // LANES, build_ids, 0)

    plsc.subcore_barrier()

    def gather_descr(c, p, t):
        sl = pl.ds(c * CHUNK, CHUNK)
        return pltpu.make_async_copy(
            tab_sh.at[idx_v[t].at[sl]], bufs[p][t], gsems[p][t])

    def writeout_descr(c, p):
        srow = (2 * sid + p) * CHUNK
        return pltpu.make_async_copy(
            acc_sh.at[pl.ds(srow, CHUNK)],
            out_hbm.at[pl.ds(base + c * CHUNK, CHUNK)],
            wsems[p])

    def process(c, p):
        b = bufs[p]
        srow = (2 * sid + p) * CHUNK
        reg = acc_sh.at[pl.ds(srow, CHUNK)]

        gather_descr(c, p, 0).wait()
        pltpu.sync_copy(b[0], reg)

        gather_descr(c, p, 1).wait()
        pltpu.sync_copy(b[1], acc_sh.at[ids_v[p]], add=True)

        gather_descr(c, p, 2).wait()
        pltpu.sync_copy(b[2], acc_sh.at[ids_v[p]], add=True)

        wo = writeout_descr(c, p)
        wo.start()
        wo.wait()
        @pl.when(c + 2 < NUM_CHUNKS)
        def _():
            for t in range(3):
                gather_descr(c + 2, p, t).start()

    for t in range(3):
        gather_descr(0, 0, t).start()
        gather_descr(1, 1, t).start()

    def pair_body(i, carry):
        process(2 * i, 0)
        process(2 * i + 1, 1)
        return carry

    lax.fori_loop(0, NUM_CHUNKS // 2, pair_body, 0)
    process(NUM_CHUNKS - 1, 0)


def kernel(edge_attr, emb0, emb1, emb2):
    a = edge_attr.astype(jnp.int32)
    i0, i1, i2 = a[:, 0], a[:, 1], a[:, 2]
    return _bond_encoder_sc(i0, i1, i2, emb0, emb1, emb2)


# speedup vs baseline: 1.3569x; 1.3569x over previous
import functools

import jax
import jax.numpy as jnp
from jax import lax
from jax.experimental import pallas as pl
from jax.experimental.pallas import tpu as pltpu
from jax.experimental.pallas import tpu_sc as plsc

HIDDEN = 128
E = 320000
VOCAB = 100
NUM_CORES = 2
NUM_SUBCORES = 16
NUM_WORKERS = NUM_CORES * NUM_SUBCORES
PER_WORKER = E // NUM_WORKERS
CHUNK = 80
NUM_CHUNKS = PER_WORKER // CHUNK
LANES = 16

_mesh = plsc.VectorSubcoreMesh(core_axis_name="c", subcore_axis_name="s")


@functools.partial(
    pl.kernel,
    mesh=_mesh,
    out_type=jax.ShapeDtypeStruct((E, HIDDEN), jnp.float32),
    scratch_types=[
        pltpu.VMEM((PER_WORKER,), jnp.int32),
        pltpu.VMEM((PER_WORKER,), jnp.int32),
        pltpu.VMEM((PER_WORKER,), jnp.int32),
        pltpu.VMEM((CHUNK,), jnp.int32),
        pltpu.VMEM((CHUNK,), jnp.int32),
        pltpu.VMEM((CHUNK, HIDDEN), jnp.float32),
        pltpu.VMEM((CHUNK, HIDDEN), jnp.float32),
        pltpu.VMEM((CHUNK, HIDDEN), jnp.float32),
        pltpu.VMEM((CHUNK, HIDDEN), jnp.float32),
        pltpu.VMEM((CHUNK, HIDDEN), jnp.float32),
        pltpu.VMEM((CHUNK, HIDDEN), jnp.float32),
        pltpu.VMEM_SHARED((3 * VOCAB, HIDDEN), jnp.float32),
        pltpu.VMEM_SHARED((2 * NUM_SUBCORES * CHUNK, HIDDEN), jnp.float32),
        pltpu.SemaphoreType.DMA,
        pltpu.SemaphoreType.DMA,
        pltpu.SemaphoreType.DMA,
        pltpu.SemaphoreType.DMA,
        pltpu.SemaphoreType.DMA,
        pltpu.SemaphoreType.DMA,
        pltpu.SemaphoreType.DMA,
        pltpu.SemaphoreType.DMA,
    ],
)
def _bond_encoder_sc(i0_hbm, i1_hbm, i2_hbm, t0_hbm, t1_hbm, t2_hbm,
                     out_hbm, i0_v, i1_v, i2_v, ids0_v, ids1_v,
                     a0_v, a1_v, a2_v, c0_v, c1_v, c2_v,
                     tab_sh, acc_sh, ga0, ga1, ga2, gb0, gb1, gb2, w0, w1):
    sid = lax.axis_index("s")
    wid = sid * NUM_CORES + lax.axis_index("c")
    base = wid * PER_WORKER

    bufs = ((a0_v, a1_v, a2_v), (c0_v, c1_v, c2_v))
    gsems = ((ga0, ga1, ga2), (gb0, gb1, gb2))
    wsems = (w0, w1)
    idx_v = (i0_v, i1_v, i2_v)
    ids_v = (ids0_v, ids1_v)

    @pl.when(sid == 0)
    def _stage():
        pltpu.sync_copy(t0_hbm, tab_sh.at[pl.ds(0, VOCAB)])
        pltpu.sync_copy(t1_hbm, tab_sh.at[pl.ds(VOCAB, VOCAB)])
        pltpu.sync_copy(t2_hbm, tab_sh.at[pl.ds(2 * VOCAB, VOCAB)])

    pltpu.sync_copy(i0_hbm.at[pl.ds(base, PER_WORKER)], i0_v)
    pltpu.sync_copy(i1_hbm.at[pl.ds(base, PER_WORKER)], i1_v)
    pltpu.sync_copy(i2_hbm.at[pl.ds(base, PER_WORKER)], i2_v)

    def rebase(j, carry):
        sl = pl.ds(j * LANES, LANES)
        i1_v[sl] = i1_v[sl] + VOCAB
        i2_v[sl] = i2_v[sl] + 2 * VOCAB
        return carry

    lax.fori_loop(0, PER_WORKER // LANES, rebase, 0)

    def build_ids(j, carry):
        sl = pl.ds(j * LANES, LANES)
        lane = lax.iota(jnp.int32, LANES) + j * LANES
        ids0_v[sl] = lane + (2 * sid) * CHUNK
        ids1_v[sl] = lane + (2 * sid + 1) * CHUNK
        return carry

    lax.fori_loop(0, CHUNK // LANES, build_ids, 0)

    plsc.subcore_barrier()

    def gather_descr(c, p, t):
        sl = pl.ds(c * CHUNK, CHUNK)
        return pltpu.make_async_copy(
            tab_sh.at[idx_v[t].at[sl]], bufs[p][t], gsems[p][t])

    def writeout_descr(c, p):
        srow = (2 * sid + p) * CHUNK
        return pltpu.make_async_copy(
            acc_sh.at[pl.ds(srow, CHUNK)],
            out_hbm.at[pl.ds(base + c * CHUNK, CHUNK)],
            wsems[p])

    def process(c, p):
        b = bufs[p]
        srow = (2 * sid + p) * CHUNK
        reg = acc_sh.at[pl.ds(srow, CHUNK)]

        @pl.when(c >= 2)
        def _():
            writeout_descr(c - 2, p).wait()
        for t in range(3):
            gather_descr(c, p, t).start()

        gather_descr(c, p, 0).wait()
        pltpu.sync_copy(b[0], reg)

        gather_descr(c, p, 1).wait()
        pltpu.sync_copy(b[1], acc_sh.at[ids_v[p]], add=True)

        gather_descr(c, p, 2).wait()
        pltpu.sync_copy(b[2], acc_sh.at[ids_v[p]], add=True)

        writeout_descr(c, p).start()

    def pair_body(i, carry):
        process(2 * i, 0)
        process(2 * i + 1, 1)
        return carry

    lax.fori_loop(0, NUM_CHUNKS // 2, pair_body, 0)
    process(NUM_CHUNKS - 1, 0)

    writeout_descr(NUM_CHUNKS - 1, 0).wait()
    writeout_descr(NUM_CHUNKS - 2, 1).wait()


def kernel(edge_attr, emb0, emb1, emb2):
    a = edge_attr.astype(jnp.int32)
    i0, i1, i2 = a[:, 0], a[:, 1], a[:, 2]
    return _bond_encoder_sc(i0, i1, i2, emb0, emb1, emb2)


# speedup vs baseline: 1.3673x; 1.0077x over previous
import functools

import jax
import jax.numpy as jnp
from jax import lax
from jax.experimental import pallas as pl
from jax.experimental.pallas import tpu as pltpu
from jax.experimental.pallas import tpu_sc as plsc

HIDDEN = 128
E = 320000
VOCAB = 100
NUM_CORES = 2
NUM_SUBCORES = 16
NUM_WORKERS = NUM_CORES * NUM_SUBCORES
PER_WORKER = E // NUM_WORKERS
CHUNK = 80
NUM_CHUNKS = PER_WORKER // CHUNK
LANES = 16

_mesh = plsc.VectorSubcoreMesh(core_axis_name="c", subcore_axis_name="s")


@functools.partial(
    pl.kernel,
    mesh=_mesh,
    out_type=jax.ShapeDtypeStruct((E, HIDDEN), jnp.float32),
    scratch_types=[
        pltpu.VMEM((PER_WORKER,), jnp.int32),
        pltpu.VMEM((PER_WORKER,), jnp.int32),
        pltpu.VMEM((PER_WORKER,), jnp.int32),
        pltpu.VMEM((CHUNK,), jnp.int32),
        pltpu.VMEM((CHUNK,), jnp.int32),
        pltpu.VMEM((CHUNK, HIDDEN), jnp.float32),
        pltpu.VMEM((CHUNK, HIDDEN), jnp.float32),
        pltpu.VMEM((CHUNK, HIDDEN), jnp.float32),
        pltpu.VMEM((CHUNK, HIDDEN), jnp.float32),
        pltpu.VMEM((CHUNK, HIDDEN), jnp.float32),
        pltpu.VMEM((CHUNK, HIDDEN), jnp.float32),
        pltpu.VMEM_SHARED((VOCAB, HIDDEN), jnp.float32),
        pltpu.VMEM_SHARED((VOCAB, HIDDEN), jnp.float32),
        pltpu.VMEM_SHARED((VOCAB, HIDDEN), jnp.float32),
        pltpu.VMEM_SHARED((2 * NUM_SUBCORES * CHUNK, HIDDEN), jnp.float32),
        pltpu.SemaphoreType.DMA,
        pltpu.SemaphoreType.DMA,
        pltpu.SemaphoreType.DMA,
        pltpu.SemaphoreType.DMA,
        pltpu.SemaphoreType.DMA,
        pltpu.SemaphoreType.DMA,
        pltpu.SemaphoreType.DMA,
        pltpu.SemaphoreType.DMA,
    ],
)
def _bond_encoder_sc(i0_hbm, i1_hbm, i2_hbm, t0_hbm, t1_hbm, t2_hbm,
                     out_hbm, i0_v, i1_v, i2_v, ids0_v, ids1_v,
                     a0_v, a1_v, a2_v, c0_v, c1_v, c2_v,
                     tab0_sh, tab1_sh, tab2_sh, acc_sh,
                     ga0, ga1, ga2, gb0, gb1, gb2, w0, w1):
    sid = lax.axis_index("s")
    wid = sid * NUM_CORES + lax.axis_index("c")
    base = wid * PER_WORKER

    bufs = ((a0_v, a1_v, a2_v), (c0_v, c1_v, c2_v))
    gsems = ((ga0, ga1, ga2), (gb0, gb1, gb2))
    wsems = (w0, w1)
    idx_v = (i0_v, i1_v, i2_v)
    tabs = (tab0_sh, tab1_sh, tab2_sh)
    ids_v = (ids0_v, ids1_v)

    @pl.when(sid == 0)
    def _stage():
        pltpu.sync_copy(t0_hbm, tab0_sh)
        pltpu.sync_copy(t1_hbm, tab1_sh)
        pltpu.sync_copy(t2_hbm, tab2_sh)

    pltpu.sync_copy(i0_hbm.at[pl.ds(base, PER_WORKER)], i0_v)
    pltpu.sync_copy(i1_hbm.at[pl.ds(base, PER_WORKER)], i1_v)
    pltpu.sync_copy(i2_hbm.at[pl.ds(base, PER_WORKER)], i2_v)

    def build_ids(j, carry):
        sl = pl.ds(j * LANES, LANES)
        lane = lax.iota(jnp.int32, LANES) + j * LANES
        ids0_v[sl] = lane + (2 * sid) * CHUNK
        ids1_v[sl] = lane + (2 * sid + 1) * CHUNK
        return carry

    lax.fori_loop(0, CHUNK // LANES, build_ids, 0)

    plsc.subcore_barrier()

    def gather_descr(c, p, t):
        sl = pl.ds(c * CHUNK, CHUNK)
        return pltpu.make_async_copy(
            tabs[t].at[idx_v[t].at[sl]], bufs[p][t], gsems[p][t])

    def writeout_descr(c, p):
        srow = (2 * sid + p) * CHUNK
        return pltpu.make_async_copy(
            acc_sh.at[pl.ds(srow, CHUNK)],
            out_hbm.at[pl.ds(base + c * CHUNK, CHUNK)],
            wsems[p])

    def process(c, p):
        b = bufs[p]
        srow = (2 * sid + p) * CHUNK
        reg = acc_sh.at[pl.ds(srow, CHUNK)]

        @pl.when(c >= 2)
        def _():
            writeout_descr(c - 2, p).wait()
        for t in range(3):
            gather_descr(c, p, t).start()

        gather_descr(c, p, 0).wait()
        pltpu.sync_copy(b[0], reg)

        gather_descr(c, p, 1).wait()
        pltpu.sync_copy(b[1], acc_sh.at[ids_v[p]], add=True)

        gather_descr(c, p, 2).wait()
        pltpu.sync_copy(b[2], acc_sh.at[ids_v[p]], add=True)

        writeout_descr(c, p).start()

    def pair_body(i, carry):
        process(2 * i, 0)
        process(2 * i + 1, 1)
        return carry

    lax.fori_loop(0, NUM_CHUNKS // 2, pair_body, 0)
    process(NUM_CHUNKS - 1, 0)

    writeout_descr(NUM_CHUNKS - 1, 0).wait()
    writeout_descr(NUM_CHUNKS - 2, 1).wait()


def kernel(edge_attr, emb0, emb1, emb2):
    a = edge_attr.astype(jnp.int32)
    i0, i1, i2 = a[:, 0], a[:, 1], a[:, 2]
    return _bond_encoder_sc(i0, i1, i2, emb0, emb1, emb2)
